# baseline (speedup 1.0000x reference)
import jax
import jax.numpy as jnp
from jax import lax
from jax.experimental import pallas as pl
from jax.experimental.pallas import tpu as pltpu, tpu_sc as plsc

N_NODES = 10000
N_EDGES = 320000
D_FEAT = 128
N_CLASSES = 21
C_OUT = 21
C_PAD = 32

NC, NS = 2, 16
NW = NC * NS
PER_W = N_EDGES // NW
MINOR = 80
KROWS = 25
CHUNK = KROWS * MINOR
NSTEPS = PER_W // CHUNK
OUT_MAJ = N_EDGES // MINOR


def _proj_body(emb_ref, wt_ref, b_ref, out_ref):
    out_ref[...] = jnp.dot(
        emb_ref[...], wt_ref[...], preferred_element_type=jnp.float32
    ) + b_ref[0][None, :]


def _project(emb, wt, b2d):
    return pl.pallas_call(
        _proj_body,
        out_shape=jax.ShapeDtypeStruct((N_NODES, C_PAD), jnp.float32),
        grid=(5,),
        in_specs=[
            pl.BlockSpec((N_NODES // 5, D_FEAT), lambda i: (i, 0)),
            pl.BlockSpec((D_FEAT, C_PAD), lambda i: (0, 0)),
            pl.BlockSpec((8, C_PAD), lambda i: (0, 0)),
        ],
        out_specs=pl.BlockSpec((N_NODES // 5, C_PAD), lambda i: (i, 0)),
    )(emb, wt, b2d)


_LANE = 16
_PACK_STEPS = CHUNK * C_OUT // _LANE


def _sc_body(p_hbm, src_hbm, dst_hbm, out_hbm, idx_s, idx_d, rows, tbuf, sem):
    wid = lax.axis_index("s") * NC + lax.axis_index("c")
    for i in range(NSTEPS):
        base = wid * PER_W + i * CHUNK
        pltpu.sync_copy(src_hbm.at[pl.ds(base, CHUNK)], idx_s)
        pltpu.sync_copy(dst_hbm.at[pl.ds(base, CHUNK)], idx_d)
        pltpu.async_copy(p_hbm.at[idx_s], rows, sem).wait()
        pltpu.async_copy(p_hbm.at[idx_d], rows, sem, add=True).wait()

        for c in range(C_OUT):
            def tpose(kb, carry, c=c):
                b16 = kb * (5 * _LANE)
                for j in range(5):
                    ve = lax.iota(jnp.int32, _LANE) + (b16 + j * _LANE)
                    vc = jnp.full((_LANE,), c, jnp.int32)
                    v = plsc.load_gather(rows, [ve, vc])
                    tbuf[c, pl.ds(b16 + j * _LANE, _LANE)] = v
                return carry

            lax.fori_loop(0, CHUNK // _LANE // 5, tpose, 0)
        pltpu.sync_copy(tbuf, out_hbm.at[:, pl.ds(base, CHUNK)])


def _edge_logits(p, src, dst):
    mesh = plsc.VectorSubcoreMesh(
        core_axis_name="c", subcore_axis_name="s",
        num_cores=NC, num_subcores=NS,
    )
    return pl.kernel(
        _sc_body,
        out_type=jax.ShapeDtypeStruct((C_OUT, N_EDGES), jnp.float32),
        mesh=mesh,
        scratch_types=[
            pltpu.VMEM((CHUNK,), jnp.int32),
            pltpu.VMEM((CHUNK,), jnp.int32),
            pltpu.VMEM((CHUNK, C_PAD), jnp.float32),
            pltpu.VMEM((C_OUT, CHUNK), jnp.float32),
            pltpu.SemaphoreType.DMA,
        ],
        compiler_params=pltpu.CompilerParams(
            use_tc_tiling_on_sc=False, needs_layout_passes=False
        ),
    )(p, src, dst)


def kernel(node_embedding, x, edge_index, W, b):
    wt = jnp.zeros((D_FEAT, C_PAD), jnp.float32).at[:, :N_CLASSES].set(W.T)
    b2d = jnp.broadcast_to(
        jnp.pad(0.5 * b, (0, C_PAD - N_CLASSES))[None, :], (8, C_PAD)
    )
    p = _project(node_embedding, wt, b2d)
    return _edge_logits(p, edge_index[0], edge_index[1]).T


# speedup vs baseline: 1.0280x; 1.0280x over previous
import jax
import jax.numpy as jnp
from jax import lax
from jax.experimental import pallas as pl
from jax.experimental.pallas import tpu as pltpu, tpu_sc as plsc

N_NODES = 10000
N_EDGES = 320000
D_FEAT = 128
N_CLASSES = 21
C_OUT = 21
C_PAD = 32

NC, NS = 2, 16
NW = NC * NS
PER_W = N_EDGES // NW
MINOR = 80
KROWS = 25
CHUNK = KROWS * MINOR
NSTEPS = PER_W // CHUNK
OUT_MAJ = N_EDGES // MINOR


def _proj_body(emb_ref, wt_ref, b_ref, out_ref):
    out_ref[...] = jnp.dot(
        emb_ref[...], wt_ref[...], preferred_element_type=jnp.float32
    ) + b_ref[0][None, :]


def _project(emb, wt, b2d):
    return pl.pallas_call(
        _proj_body,
        out_shape=jax.ShapeDtypeStruct((N_NODES, C_PAD), jnp.float32),
        grid=(5,),
        in_specs=[
            pl.BlockSpec((N_NODES // 5, D_FEAT), lambda i: (i, 0)),
            pl.BlockSpec((D_FEAT, C_PAD), lambda i: (0, 0)),
            pl.BlockSpec((8, C_PAD), lambda i: (0, 0)),
        ],
        out_specs=pl.BlockSpec((N_NODES // 5, C_PAD), lambda i: (i, 0)),
    )(emb, wt, b2d)


_LANE = 16
_PACK_STEPS = CHUNK * C_OUT // _LANE


def _sc_body(p_hbm, src_hbm, dst_hbm, out_hbm, idx_s, idx_d, rows, tbuf, sem):
    wid = lax.axis_index("s") * NC + lax.axis_index("c")
    for i in range(NSTEPS):
        base = wid * PER_W + i * CHUNK
        pltpu.sync_copy(src_hbm.at[pl.ds(base, CHUNK)], idx_s)
        pltpu.sync_copy(dst_hbm.at[pl.ds(base, CHUNK)], idx_d)
        pltpu.async_copy(p_hbm.at[idx_s], rows, sem).wait()
        pltpu.async_copy(p_hbm.at[idx_d], rows, sem, add=True).wait()

        def tpose(k, carry):
            b = k * _LANE
            ve = lax.iota(jnp.int32, _LANE) + b
            for c in range(C_OUT):
                vc = jnp.full((_LANE,), c, jnp.int32)
                v = plsc.load_gather(rows, [ve, vc])
                tbuf[c, pl.ds(b, _LANE)] = v
            return carry

        lax.fori_loop(0, CHUNK // _LANE, tpose, 0)
        pltpu.sync_copy(tbuf, out_hbm.at[:, pl.ds(base, CHUNK)])


def _edge_logits(p, src, dst):
    mesh = plsc.VectorSubcoreMesh(
        core_axis_name="c", subcore_axis_name="s",
        num_cores=NC, num_subcores=NS,
    )
    return pl.kernel(
        _sc_body,
        out_type=jax.ShapeDtypeStruct((C_OUT, N_EDGES), jnp.float32),
        mesh=mesh,
        scratch_types=[
            pltpu.VMEM((CHUNK,), jnp.int32),
            pltpu.VMEM((CHUNK,), jnp.int32),
            pltpu.VMEM((CHUNK, C_PAD), jnp.float32),
            pltpu.VMEM((C_OUT, CHUNK), jnp.float32),
            pltpu.SemaphoreType.DMA,
        ],
        compiler_params=pltpu.CompilerParams(
            use_tc_tiling_on_sc=False, needs_layout_passes=False
        ),
    )(p, src, dst)


def kernel(node_embedding, x, edge_index, W, b):
    wt = jnp.zeros((D_FEAT, C_PAD), jnp.float32).at[:, :N_CLASSES].set(W.T)
    b2d = jnp.broadcast_to(
        jnp.pad(0.5 * b, (0, C_PAD - N_CLASSES))[None, :], (8, C_PAD)
    )
    p = _project(node_embedding, wt, b2d)
    return _edge_logits(p, edge_index[0], edge_index[1]).T


# speedup vs baseline: 1.3400x; 1.3034x over previous
import jax
import jax.numpy as jnp
from jax import lax
from jax.experimental import pallas as pl
from jax.experimental.pallas import tpu as pltpu, tpu_sc as plsc

N_NODES = 10000
N_EDGES = 320000
D_FEAT = 128
N_CLASSES = 21
C_OUT = 21
C_PAD = 32

NC, NS = 2, 16
NW = NC * NS
PER_W = N_EDGES // NW
MINOR = 80
KROWS = 25
CHUNK = KROWS * MINOR
NSTEPS = PER_W // CHUNK
OUT_MAJ = N_EDGES // MINOR


def _proj_body(emb_ref, wt_ref, b_ref, out_ref):
    out_ref[...] = jnp.dot(
        emb_ref[...], wt_ref[...], preferred_element_type=jnp.float32
    ) + b_ref[0][None, :]


def _project(emb, wt, b2d):
    return pl.pallas_call(
        _proj_body,
        out_shape=jax.ShapeDtypeStruct((N_NODES, C_PAD), jnp.float32),
        grid=(5,),
        in_specs=[
            pl.BlockSpec((N_NODES // 5, D_FEAT), lambda i: (i, 0)),
            pl.BlockSpec((D_FEAT, C_PAD), lambda i: (0, 0)),
            pl.BlockSpec((8, C_PAD), lambda i: (0, 0)),
        ],
        out_specs=pl.BlockSpec((N_NODES // 5, C_PAD), lambda i: (i, 0)),
    )(emb, wt, b2d)


_LANE = 16
_PACK_STEPS = CHUNK * C_OUT // _LANE


def _sc_body(p_hbm, src_hbm, dst_hbm, out_hbm, idx_s, idx_d, rows, tbuf, sem):
    wid = lax.axis_index("s") * NC + lax.axis_index("c")
    for i in range(NSTEPS):
        base = wid * PER_W + i * CHUNK
        pltpu.sync_copy(src_hbm.at[pl.ds(base, CHUNK)], idx_s)
        pltpu.sync_copy(dst_hbm.at[pl.ds(base, CHUNK)], idx_d)
        pltpu.async_copy(p_hbm.at[idx_s], rows, sem).wait()
        pltpu.async_copy(p_hbm.at[idx_d], rows, sem, add=True).wait()

        def tpose(k, carry):
            b = k * _LANE
            ve = lax.iota(jnp.int32, _LANE) + b
            vs = [
                plsc.load_gather(rows, [ve, jnp.full((_LANE,), c, jnp.int32)])
                for c in range(C_OUT)
            ]
            for c in range(C_OUT):
                tbuf[c, pl.ds(b, _LANE)] = vs[c]
            return carry

        lax.fori_loop(0, CHUNK // _LANE, tpose, 0)
        pltpu.sync_copy(tbuf, out_hbm.at[:, pl.ds(base, CHUNK)])


def _edge_logits(p, src, dst):
    mesh = plsc.VectorSubcoreMesh(
        core_axis_name="c", subcore_axis_name="s",
        num_cores=NC, num_subcores=NS,
    )
    return pl.kernel(
        _sc_body,
        out_type=jax.ShapeDtypeStruct((C_OUT, N_EDGES), jnp.float32),
        mesh=mesh,
        scratch_types=[
            pltpu.VMEM((CHUNK,), jnp.int32),
            pltpu.VMEM((CHUNK,), jnp.int32),
            pltpu.VMEM((CHUNK, C_PAD), jnp.float32),
            pltpu.VMEM((C_OUT, CHUNK), jnp.float32),
            pltpu.SemaphoreType.DMA,
        ],
        compiler_params=pltpu.CompilerParams(
            use_tc_tiling_on_sc=False, needs_layout_passes=False
        ),
    )(p, src, dst)


def kernel(node_embedding, x, edge_index, W, b):
    wt = jnp.zeros((D_FEAT, C_PAD), jnp.float32).at[:, :N_CLASSES].set(W.T)
    b2d = jnp.broadcast_to(
        jnp.pad(0.5 * b, (0, C_PAD - N_CLASSES))[None, :], (8, C_PAD)
    )
    p = _project(node_embedding, wt, b2d)
    return _edge_logits(p, edge_index[0], edge_index[1]).T


# speedup vs baseline: 1.3816x; 1.0311x over previous
import jax
import jax.numpy as jnp
from jax import lax
from jax.experimental import pallas as pl
from jax.experimental.pallas import tpu as pltpu, tpu_sc as plsc

N_NODES = 10000
N_EDGES = 320000
D_FEAT = 128
N_CLASSES = 21
C_OUT = 21
C_PAD = 32

NC, NS = 2, 16
NW = NC * NS
PER_W = N_EDGES // NW
CHUNK = 400
NSTEPS = PER_W // CHUNK


def _proj_body(emb_ref, wt_ref, b_ref, out_ref):
    out_ref[...] = jnp.dot(
        emb_ref[...], wt_ref[...], preferred_element_type=jnp.float32
    ) + b_ref[0][None, :]


def _project(emb, wt, b2d):
    return pl.pallas_call(
        _proj_body,
        out_shape=jax.ShapeDtypeStruct((N_NODES, C_PAD), jnp.float32),
        grid=(5,),
        in_specs=[
            pl.BlockSpec((N_NODES // 5, D_FEAT), lambda i: (i, 0)),
            pl.BlockSpec((D_FEAT, C_PAD), lambda i: (0, 0)),
            pl.BlockSpec((8, C_PAD), lambda i: (0, 0)),
        ],
        out_specs=pl.BlockSpec((N_NODES // 5, C_PAD), lambda i: (i, 0)),
    )(emb, wt, b2d)


_LANE = 16
_PACK_STEPS = CHUNK * C_OUT // _LANE


def _sc_body(p_hbm, src_hbm, dst_hbm, out_hbm,
             idx_s, idx_d, rows_ab, tbuf_ab, sem_ab, semw_ab):
    wid = lax.axis_index("s") * NC + lax.axis_index("c")
    off = wid * PER_W
    pltpu.sync_copy(src_hbm.at[pl.ds(off, PER_W)], idx_s)
    pltpu.sync_copy(dst_hbm.at[pl.ds(off, PER_W)], idx_d)

    def transpose_chunk(rows, tbuf):
        def tpose(k, carry):
            b = k * _LANE
            ve = lax.iota(jnp.int32, _LANE) + b
            vs = [
                plsc.load_gather(rows, [ve, jnp.full((_LANE,), c, jnp.int32)])
                for c in range(C_OUT)
            ]
            for c in range(C_OUT):
                tbuf[c, pl.ds(b, _LANE)] = vs[c]
            return carry

        lax.fori_loop(0, CHUNK // _LANE, tpose, 0)

    wb = [None, None]
    d = pltpu.async_copy(p_hbm.at[idx_s.at[pl.ds(0, CHUNK)]], rows_ab[0], sem_ab[0])
    for i in range(NSTEPS):
        par = i & 1
        d.wait()
        add_d = pltpu.async_copy(
            p_hbm.at[idx_d.at[pl.ds(i * CHUNK, CHUNK)]],
            rows_ab[par], sem_ab[par], add=True,
        )
        if i + 1 < NSTEPS:
            d = pltpu.async_copy(
                p_hbm.at[idx_s.at[pl.ds((i + 1) * CHUNK, CHUNK)]],
                rows_ab[1 - par], sem_ab[1 - par],
            )
        add_d.wait()
        if wb[par] is not None:
            wb[par].wait()
        transpose_chunk(rows_ab[par], tbuf_ab[par])
        wb[par] = pltpu.async_copy(
            tbuf_ab[par], out_hbm.at[:, pl.ds(off + i * CHUNK, CHUNK)], semw_ab[par]
        )
    for par in (0, 1):
        if wb[par] is not None:
            wb[par].wait()


def _edge_logits(p, src, dst):
    mesh = plsc.VectorSubcoreMesh(
        core_axis_name="c", subcore_axis_name="s",
        num_cores=NC, num_subcores=NS,
    )
    return pl.kernel(
        _sc_body,
        out_type=jax.ShapeDtypeStruct((C_OUT, N_EDGES), jnp.float32),
        mesh=mesh,
        scratch_types=[
            pltpu.VMEM((PER_W,), jnp.int32),
            pltpu.VMEM((PER_W,), jnp.int32),
            [pltpu.VMEM((CHUNK, C_PAD), jnp.float32)] * 2,
            [pltpu.VMEM((C_OUT, CHUNK), jnp.float32)] * 2,
            [pltpu.SemaphoreType.DMA] * 2,
            [pltpu.SemaphoreType.DMA] * 2,
        ],
        compiler_params=pltpu.CompilerParams(
            use_tc_tiling_on_sc=False, needs_layout_passes=False
        ),
    )(p, src, dst)


def kernel(node_embedding, x, edge_index, W, b):
    wt = jnp.zeros((D_FEAT, C_PAD), jnp.float32).at[:, :N_CLASSES].set(W.T)
    b2d = jnp.broadcast_to(
        jnp.pad(0.5 * b, (0, C_PAD - N_CLASSES))[None, :], (8, C_PAD)
    )
    p = _project(node_embedding, wt, b2d)
    return _edge_logits(p, edge_index[0], edge_index[1]).T
